# baseline (speedup 1.0000x reference)
import functools

import jax
import jax.numpy as jnp
from jax import lax
from jax.experimental import pallas as pl
from jax.experimental.pallas import tpu as pltpu
from jax.experimental.pallas import tpu_sc as plsc

BATCH = 16384
D = 2048
L = 16
NC, NS = 2, 16
NW = NC * NS
ROWS_PER_W = BATCH // NW
RB = 8
NBLK = ROWS_PER_W // RB
NCHUNK = D // L


def _permute_body(x_hbm, perm_hbm, out_hbm, perm_v, in_v, out_v):
    wid = lax.axis_index("s") * NC + lax.axis_index("c")
    elem0 = wid * (ROWS_PER_W * D)
    pltpu.sync_copy(perm_hbm, perm_v)

    def block(b, carry):
        e0 = elem0 + b * (RB * D)
        pltpu.sync_copy(x_hbm.at[pl.ds(e0, RB * D)], in_v)

        def chunk(k, c2):
            idx = perm_v[pl.ds(k * L, L)]
            for r in range(RB):
                out_v[pl.ds(k * L + r * D, L)] = plsc.load_gather(
                    in_v, [idx + r * D])
            return c2

        lax.fori_loop(0, NCHUNK, chunk, 0, unroll=False)
        pltpu.sync_copy(out_v, out_hbm.at[pl.ds(e0, RB * D)])
        return carry

    lax.fori_loop(0, NBLK, block, 0, unroll=False)


@functools.partial(
    pl.kernel,
    out_type=jax.ShapeDtypeStruct((BATCH * D,), jnp.float32),
    mesh=plsc.VectorSubcoreMesh(core_axis_name="c", subcore_axis_name="s"),
    scratch_types=[
        pltpu.VMEM((D,), jnp.int32),
        pltpu.VMEM((RB * D,), jnp.float32),
        pltpu.VMEM((RB * D,), jnp.float32),
    ],
    compiler_params=pltpu.CompilerParams(needs_layout_passes=False),
)
def _permute(x_hbm, perm_hbm, out_hbm, perm_v, in_v, out_v):
    _permute_body(x_hbm, perm_hbm, out_hbm, perm_v, in_v, out_v)


def kernel(x, permutation):
    out = _permute(x.reshape(-1), permutation.astype(jnp.int32))
    return out.reshape(BATCH, D)


# speedup vs baseline: 1.2357x; 1.2357x over previous
import functools

import jax
import jax.numpy as jnp
from jax import lax
from jax.experimental import pallas as pl
from jax.experimental.pallas import tpu as pltpu
from jax.experimental.pallas import tpu_sc as plsc

BATCH = 16384
D = 2048
L = 16
NC, NS = 2, 16
NW = NC * NS
ROWS_PER_W = BATCH // NW
RB = 8
BLK = RB * D
NBLK = ROWS_PER_W // RB
NCHUNK = D // L
NBUF = 2


def _permute_body(x_hbm, perm_hbm, out_hbm, perm_v,
                  in0, in1, out0, out1, isem0, isem1, osem0, osem1):
    ins, outs = [in0, in1], [out0, out1]
    isems, osems = [isem0, isem1], [osem0, osem1]

    wid = lax.axis_index("s") * NC + lax.axis_index("c")
    elem0 = wid * (ROWS_PER_W * D)
    pltpu.sync_copy(perm_hbm, perm_v)

    def start_in(b, s):
        pltpu.async_copy(x_hbm.at[pl.ds(elem0 + b * BLK, BLK)], ins[s], isems[s])

    def start_out(b, s):
        pltpu.async_copy(outs[s], out_hbm.at[pl.ds(elem0 + b * BLK, BLK)], osems[s])

    def wait_in(s):
        pltpu.make_async_copy(x_hbm.at[pl.ds(0, BLK)], ins[s], isems[s]).wait()

    def wait_out(s):
        pltpu.make_async_copy(outs[s], out_hbm.at[pl.ds(0, BLK)], osems[s]).wait()

    for s in range(NBUF):
        start_in(s, s)

    def outer(b0, carry):
        for s in range(NBUF):
            b = b0 + s
            wait_in(s)

            @pl.when(b0 > 0)
            def _():
                wait_out(s)

            def chunk(k, c2):
                idx = perm_v[pl.ds(k * L, L)]
                for r in range(RB):
                    outs[s][pl.ds(k * L + r * D, L)] = plsc.load_gather(
                        ins[s], [idx + r * D])
                return c2

            lax.fori_loop(0, NCHUNK, chunk, 0, unroll=2)
            start_out(b, s)

            @pl.when(b0 < NBLK - NBUF)
            def _():
                start_in(b + NBUF, s)
        return carry

    lax.fori_loop(0, NBLK // NBUF, lambda i, c: outer(i * NBUF, c), 0,
                  unroll=False)
    for s in range(NBUF):
        wait_out(s)


@functools.partial(
    pl.kernel,
    out_type=jax.ShapeDtypeStruct((BATCH * D,), jnp.float32),
    mesh=plsc.VectorSubcoreMesh(core_axis_name="c", subcore_axis_name="s"),
    scratch_types=[
        pltpu.VMEM((D,), jnp.int32),
        pltpu.VMEM((BLK,), jnp.float32),
        pltpu.VMEM((BLK,), jnp.float32),
        pltpu.VMEM((BLK,), jnp.float32),
        pltpu.VMEM((BLK,), jnp.float32),
        pltpu.SemaphoreType.DMA,
        pltpu.SemaphoreType.DMA,
        pltpu.SemaphoreType.DMA,
        pltpu.SemaphoreType.DMA,
    ],
    compiler_params=pltpu.CompilerParams(needs_layout_passes=False),
)
def _permute(x_hbm, perm_hbm, out_hbm, perm_v,
             in0, in1, out0, out1, isem0, isem1, osem0, osem1):
    _permute_body(x_hbm, perm_hbm, out_hbm, perm_v,
                  in0, in1, out0, out1, isem0, isem1, osem0, osem1)


def kernel(x, permutation):
    out = _permute(x.reshape(-1), permutation.astype(jnp.int32))
    return out.reshape(BATCH, D)
